# Initial kernel scaffold; baseline (speedup 1.0000x reference)
import jax
import jax.numpy as jnp
from jax.experimental import pallas as pl


def kernel(embeddings, target):
    raise NotImplementedError("write your pallas kernel here")


# speedup vs baseline: 556.3921x; 556.3921x over previous
import jax
import jax.numpy as jnp
from jax.experimental import pallas as pl

_MARGIN = 1.0


def _triplet_loss_kernel(emb_ref, lab_row_ref, lab_col_ref, out_ref):
    emb = emb_ref[...]
    n = emb.shape[0]
    g = jax.lax.dot_general(
        emb, emb, (((1,), (1,)), ((), ())),
        preferred_element_type=jnp.float32,
    )
    row = jax.lax.broadcasted_iota(jnp.int32, (n, n), 0)
    col = jax.lax.broadcasted_iota(jnp.int32, (n, n), 1)
    eye = row == col
    gd = jnp.where(eye, g, 0.0)
    d_col = jnp.sum(gd, axis=1, keepdims=True)
    d_row = jnp.sum(gd, axis=0, keepdims=True)
    dist = jnp.maximum(d_col + d_row - 2.0 * g, 0.0)

    same = lab_col_ref[...] == lab_row_ref[...]
    dneg = jnp.where(same, jnp.float32(jnp.inf), dist)
    mn = jnp.min(dneg, axis=1, keepdims=True)
    dn = jnp.where(jnp.isinf(mn), dist[:, 0:1], mn)

    valid = same & (row < col)
    losses = jnp.maximum(dist - dn + _MARGIN, 0.0)
    loss_sum = jnp.sum(jnp.where(valid, losses, 0.0), keepdims=True)
    cnt = jnp.sum(valid.astype(jnp.float32), keepdims=True)
    out_ref[...] = (loss_sum / cnt).reshape(1, 1)


def kernel(embeddings, target):
    b = embeddings.shape[0]
    lab = target.astype(jnp.int32)
    out = pl.pallas_call(
        _triplet_loss_kernel,
        out_shape=jax.ShapeDtypeStruct((1, 1), jnp.float32),
    )(embeddings, lab.reshape(1, b), lab.reshape(b, 1))
    return out[0, 0]
